# baseline (speedup 1.0000x reference)
import functools

import jax
import jax.numpy as jnp
from jax import lax
from jax.experimental import pallas as pl
from jax.experimental.pallas import tpu as pltpu
from jax.experimental.pallas import tpu_sc as plsc

_B, _S, _D, _HD, _K, _A, _NIT = 32, 7, 64, 128, 3, 50, 3
_NEG = -1e30
_BF = jnp.bfloat16


def _dot(a, b):
    return jax.lax.dot(a.astype(_BF), b.astype(_BF),
                       preferred_element_type=jnp.float32)


def _dot_t(a, b, dims):
    return jax.lax.dot_general(a.astype(_BF), b.astype(_BF),
                               (dims, ((), ())),
                               preferred_element_type=jnp.float32)


def _ln(x, g, b, eps=1e-5):
    m = x.mean(-1, keepdims=True)
    v = ((x - m) ** 2).mean(-1, keepdims=True)
    return (x - m) / jnp.sqrt(v + eps) * g + b


def _attnloop_body(kk_ref, vv_ref, noise_ref,
                   mu_ref, logsig_ref, ln_s_g_ref, ln_s_b_ref, wq_ref,
                   gru_wi_ref, gru_bi_ref, gru_wh_ref, gru_bh_ref,
                   ln_m_g_ref, ln_m_b_ref,
                   mlp_w1_ref, mlp_b1_ref, mlp_w2_ref, mlp_b2_ref,
                   o_ref):
    kk = kk_ref[0]
    vv = vv_ref[0]
    vvp = jnp.concatenate([vv, jnp.zeros((60, _D), jnp.float32)], axis=0)
    slots = mu_ref[...] + noise_ref[0] * jnp.exp(logsig_ref[...])
    scale = _D ** -0.5
    for _ in range(_NIT):
        prev = slots
        sn = _ln(slots, ln_s_g_ref[...], ln_s_b_ref[...])
        q = _dot(sn, wq_ref[...])
        logits = _dot_t(kk, q, ((1,), (1,))) * scale
        attn = jax.nn.softmax(logits, axis=-1)
        w = attn / (attn.sum(axis=0, keepdims=True) + 1e-8)
        wp = jnp.concatenate([w, jnp.zeros((60, _S), jnp.float32)], axis=0)
        upd = _dot_t(wp, vvp, ((0,), (0,)))
        gi = _dot(upd, gru_wi_ref[...]) + gru_bi_ref[...]
        gh = _dot(prev, gru_wh_ref[...]) + gru_bh_ref[...]
        r = jax.nn.sigmoid(gi[:, :_D] + gh[:, :_D])
        zg = jax.nn.sigmoid(gi[:, _D:2 * _D] + gh[:, _D:2 * _D])
        n = jnp.tanh(gi[:, 2 * _D:] + r * gh[:, 2 * _D:])
        slots = (1 - zg) * n + zg * prev
        mls = _ln(slots, ln_m_g_ref[...], ln_m_b_ref[...])
        h1 = jnp.maximum(_dot(mls, mlp_w1_ref[...]) + mlp_b1_ref[...], 0.0)
        slots = slots + (_dot(h1, mlp_w2_ref[...]) + mlp_b2_ref[...])
    o_ref[0] = slots


def _attn_loop(kk, vv, slot_noise, p):
    Bn, M, _ = kk.shape
    mu = p['slots_mu'].reshape(_S, _D)
    logsig = p['slots_logsigma'].reshape(_S, _D)
    bf = lambda a: a.astype(_BF)
    full = lambda a: pl.BlockSpec(a.shape, lambda i: (0,) * a.ndim)
    weights = [mu, logsig, p['ln_s_g'], p['ln_s_b'], bf(p['Wq']),
               bf(p['gru_wi']), p['gru_bi'], bf(p['gru_wh']), p['gru_bh'],
               p['ln_m_g'], p['ln_m_b'],
               bf(p['mlp_w1']), p['mlp_b1'], bf(p['mlp_w2']), p['mlp_b2']]
    return pl.pallas_call(
        _attnloop_body,
        grid=(Bn,),
        in_specs=[
            pl.BlockSpec((1, M, _D), lambda i: (i, 0, 0)),
            pl.BlockSpec((1, M, _D), lambda i: (i, 0, 0)),
            pl.BlockSpec((1, _S, _D), lambda i: (i, 0, 0)),
        ] + [full(a) for a in weights],
        out_specs=pl.BlockSpec((1, _S, _D), lambda i: (i, 0, 0)),
        out_shape=jax.ShapeDtypeStruct((Bn, _S, _D), jnp.float32),
    )(kk, vv, slot_noise, *weights)


def _score_body(slots_ref, enc_w_ref, enc_b_ref, dec_w_ref, dec_b_ref, o_ref):
    slots = slots_ref[0]
    mu = _dot(slots, enc_w_ref[...]) + enc_b_ref[...]
    cols = []
    for a in range(_A):
        rec = _dot(mu[:, a * 32:(a + 1) * 32], dec_w_ref[a]) + dec_b_ref[a]
        d = rec - slots
        cols.append(-jnp.sum(d * d, axis=-1, keepdims=True))
    cols.append(jnp.full((_S, 64 - _A), _NEG, jnp.float32))
    o_ref[0] = jnp.concatenate(cols, axis=1)


def _vae_scores(slots, p):
    Bn = slots.shape[0]
    enc_w = p['vae_enc_w'].transpose(1, 0, 2).reshape(_D, _A * 32)
    enc_w = enc_w.astype(_BF)
    enc_b = p['vae_enc_b'].reshape(_A * 32)
    dec_w = p['vae_dec_w'].astype(_BF)
    full = lambda a: pl.BlockSpec(a.shape, lambda i: (0,) * a.ndim)
    return pl.pallas_call(
        _score_body,
        grid=(Bn,),
        in_specs=[pl.BlockSpec((1, _S, _D), lambda i: (i, 0, 0)),
                  full(enc_w), full(enc_b),
                  full(dec_w), full(p['vae_dec_b'])],
        out_specs=pl.BlockSpec((1, _S, 64), lambda i: (i, 0, 0)),
        out_shape=jax.ShapeDtypeStruct((Bn, _S, 64), jnp.float32),
    )(slots, enc_w, enc_b, dec_w, p['vae_dec_b'])


def _topk_sc_call(scores2d):
    R0 = scores2d.shape[0]
    NWROW = 16
    R = NWROW * 16
    sc = jnp.pad(scores2d, ((0, R - R0), (0, 0)))
    sc_t = sc.T.reshape(64, NWROW, 16).transpose(1, 0, 2)
    info = plsc.get_sparse_core_info()
    mesh = plsc.VectorSubcoreMesh(core_axis_name="c", subcore_axis_name="s")

    @functools.partial(
        pl.kernel, mesh=mesh,
        out_type=jax.ShapeDtypeStruct((NWROW, _K, 16), jnp.int32),
        scratch_types=[
            pltpu.VMEM((64, 16), jnp.float32),
            pltpu.VMEM((_K, 16), jnp.int32),
        ],
    )
    def _k(scores_hbm, out_hbm, sc_v, sel_v):
        wid = lax.axis_index("s") * info.num_cores + lax.axis_index("c")

        @pl.when(wid < NWROW)
        def _():
            pltpu.sync_copy(scores_hbm.at[wid], sc_v)
            b1 = jnp.full((16,), _NEG, jnp.float32)
            b2 = jnp.full((16,), _NEG, jnp.float32)
            b3 = jnp.full((16,), _NEG, jnp.float32)
            zero = jnp.zeros((16,), jnp.int32)
            i1, i2, i3 = zero, zero, zero
            for a in range(_A):
                v = sc_v[a, :]
                gt1 = v > b1
                gt2 = v > b2
                gt3 = v > b3
                av = jnp.full((16,), a, jnp.int32)
                b3 = jnp.where(gt2, b2, jnp.where(gt3, v, b3))
                i3 = jnp.where(gt2, i2, jnp.where(gt3, av, i3))
                b2 = jnp.where(gt1, b1, jnp.where(gt2, v, b2))
                i2 = jnp.where(gt1, i1, jnp.where(gt2, av, i2))
                b1 = jnp.where(gt1, v, b1)
                i1 = jnp.where(gt1, av, i1)
            sel_v[0, :] = i1
            sel_v[1, :] = i2
            sel_v[2, :] = i3
            pltpu.sync_copy(sel_v, out_hbm.at[wid])

    out = _k(sc_t)
    return out.transpose(0, 2, 1).reshape(R, _K)[:R0]


def _expert_body(sel_ref, slots_ref, in_w_ref, in_b_ref,
                 blk_w_ref, blk_b_ref, out_w_ref, out_b_ref, o_ref):
    b = pl.program_id(0)
    slots = slots_ref[0]
    for s in range(_S):
        x = slots[s:s + 1, :]
        for k in range(_K):
            a = sel_ref[b, s, k]
            h = jnp.maximum(
                _dot(x, in_w_ref[a]) + in_b_ref[pl.ds(a, 1), :], 0.0)
            for i in range(3):
                w = blk_w_ref[pl.ds(a, 1), i, :, :].reshape(256, 256)
                bb = blk_b_ref[pl.ds(a, 1), i, :].reshape(1, 256)
                h = h + jnp.maximum(_dot(h, w) + bb, 0.0)
            out = _dot(h, out_w_ref[a]) + out_b_ref[pl.ds(a, 1), :]
            o_ref[0, pl.ds(s * _K + k, 1), :] = out
    return


def _expert_mlps(sel, slots, p):
    Bn = slots.shape[0]
    in_w = p['ag_in_w'].astype(_BF)
    blk_w = p['ag_blk_w'].astype(_BF)
    out_w = p['ag_out_w'].astype(_BF)
    full = lambda a: pl.BlockSpec(a.shape, lambda i: (0,) * a.ndim)
    return pl.pallas_call(
        _expert_body,
        grid=(Bn,),
        in_specs=[
            pl.BlockSpec(memory_space=pltpu.SMEM),
            pl.BlockSpec((1, _S, _D), lambda i: (i, 0, 0)),
            full(in_w), full(p['ag_in_b']),
            full(blk_w), full(p['ag_blk_b']),
            full(out_w), full(p['ag_out_b']),
        ],
        out_specs=pl.BlockSpec((1, _S * _K, _HD), lambda i: (i, 0, 0)),
        out_shape=jax.ShapeDtypeStruct((Bn, _S * _K, _HD), jnp.float32),
    )(sel, slots, in_w, p['ag_in_b'], blk_w, p['ag_blk_b'],
      out_w, p['ag_out_b'])


def kernel(images, slot_noise, params):
    p = params
    Bn = images.shape[0]

    def _conv(x, w, b, stride):
        y = jax.lax.conv_general_dilated(
            x, w, (stride, stride), 'SAME',
            dimension_numbers=('NCHW', 'OIHW', 'NCHW'))
        return y + b[None, :, None, None]

    x = jax.nn.relu(_conv(images, p['conv1_w'], p['conv1_b'], 2))
    x = jax.nn.relu(_conv(x, p['conv2_w'], p['conv2_b'], 2))
    x = jax.nn.relu(_conv(x, p['conv3_w'], p['conv3_b'], 2))
    x = jax.nn.relu(_conv(x, p['conv4_w'], p['conv4_b'], 2))
    x = _conv(x, p['convo_w'], p['convo_b'], 1)
    feat = x.transpose(0, 2, 3, 1).reshape(Bn, -1, x.shape[1])
    inp = _ln(feat, p['ln_in_g'], p['ln_in_b'])
    kk = inp @ p['Wk']
    vv = inp @ p['Wv']

    slots = _attn_loop(kk, vv, slot_noise, p)
    scores = _vae_scores(slots, p)
    sel = _topk_sc_call(scores.reshape(Bn * _S, 64)).reshape(Bn, _S, _K)
    out = _expert_mlps(sel, slots, p)
    return out.reshape(Bn, _S * _K * _HD)


# speedup vs baseline: 1.4986x; 1.4986x over previous
import functools

import jax
import jax.numpy as jnp
from jax import lax
from jax.experimental import pallas as pl
from jax.experimental.pallas import tpu as pltpu
from jax.experimental.pallas import tpu_sc as plsc

_B, _S, _D, _HD, _K, _A, _NIT = 32, 7, 64, 128, 3, 50, 3
_NEG = -1e30
_BF = jnp.bfloat16


def _dot(a, b):
    return jax.lax.dot(a.astype(_BF), b.astype(_BF),
                       preferred_element_type=jnp.float32)


def _dot_t(a, b, dims):
    return jax.lax.dot_general(a.astype(_BF), b.astype(_BF),
                               (dims, ((), ())),
                               preferred_element_type=jnp.float32)


def _ln(x, g, b, eps=1e-5):
    m = x.mean(-1, keepdims=True)
    v = ((x - m) ** 2).mean(-1, keepdims=True)
    return (x - m) / jnp.sqrt(v + eps) * g + b


def _attnloop_body(kk_ref, vv_ref, noise_ref,
                   mu_ref, logsig_ref, ln_s_g_ref, ln_s_b_ref, wq_ref,
                   gru_wi_ref, gru_bi_ref, gru_wh_ref, gru_bh_ref,
                   ln_m_g_ref, ln_m_b_ref,
                   mlp_w1_ref, mlp_b1_ref, mlp_w2_ref, mlp_b2_ref,
                   o_ref):
    kk = kk_ref[0]
    vv = vv_ref[0]
    vvp = jnp.concatenate([vv, jnp.zeros((60, _D), jnp.float32)], axis=0)
    slots = mu_ref[...] + noise_ref[0] * jnp.exp(logsig_ref[...])
    scale = _D ** -0.5
    for _ in range(_NIT):
        prev = slots
        sn = _ln(slots, ln_s_g_ref[...], ln_s_b_ref[...])
        q = _dot(sn, wq_ref[...])
        logits = _dot_t(kk, q, ((1,), (1,))) * scale
        attn = jax.nn.softmax(logits, axis=-1)
        w = attn / (attn.sum(axis=0, keepdims=True) + 1e-8)
        wp = jnp.concatenate([w, jnp.zeros((60, _S), jnp.float32)], axis=0)
        upd = _dot_t(wp, vvp, ((0,), (0,)))
        gi = _dot(upd, gru_wi_ref[...]) + gru_bi_ref[...]
        gh = _dot(prev, gru_wh_ref[...]) + gru_bh_ref[...]
        r = jax.nn.sigmoid(gi[:, :_D] + gh[:, :_D])
        zg = jax.nn.sigmoid(gi[:, _D:2 * _D] + gh[:, _D:2 * _D])
        n = jnp.tanh(gi[:, 2 * _D:] + r * gh[:, 2 * _D:])
        slots = (1 - zg) * n + zg * prev
        mls = _ln(slots, ln_m_g_ref[...], ln_m_b_ref[...])
        h1 = jnp.maximum(_dot(mls, mlp_w1_ref[...]) + mlp_b1_ref[...], 0.0)
        slots = slots + (_dot(h1, mlp_w2_ref[...]) + mlp_b2_ref[...])
    o_ref[0] = slots


def _attn_loop(kk, vv, slot_noise, p):
    Bn, M, _ = kk.shape
    mu = p['slots_mu'].reshape(_S, _D)
    logsig = p['slots_logsigma'].reshape(_S, _D)
    bf = lambda a: a.astype(_BF)
    full = lambda a: pl.BlockSpec(a.shape, lambda i: (0,) * a.ndim)
    weights = [mu, logsig, p['ln_s_g'], p['ln_s_b'], bf(p['Wq']),
               bf(p['gru_wi']), p['gru_bi'], bf(p['gru_wh']), p['gru_bh'],
               p['ln_m_g'], p['ln_m_b'],
               bf(p['mlp_w1']), p['mlp_b1'], bf(p['mlp_w2']), p['mlp_b2']]
    return pl.pallas_call(
        _attnloop_body,
        grid=(Bn,),
        in_specs=[
            pl.BlockSpec((1, M, _D), lambda i: (i, 0, 0)),
            pl.BlockSpec((1, M, _D), lambda i: (i, 0, 0)),
            pl.BlockSpec((1, _S, _D), lambda i: (i, 0, 0)),
        ] + [full(a) for a in weights],
        out_specs=pl.BlockSpec((1, _S, _D), lambda i: (i, 0, 0)),
        out_shape=jax.ShapeDtypeStruct((Bn, _S, _D), jnp.float32),
    )(kk, vv, slot_noise, *weights)


def _score_body(slots_ref, enc_w_ref, enc_b_ref, dec_w_ref, dec_b_ref, o_ref):
    slots = slots_ref[0]
    mu = _dot(slots, enc_w_ref[...]) + enc_b_ref[...]
    cols = []
    for a in range(_A):
        rec = _dot(mu[:, a * 32:(a + 1) * 32], dec_w_ref[a]) + dec_b_ref[a]
        d = rec - slots
        cols.append(-jnp.sum(d * d, axis=-1, keepdims=True))
    cols.append(jnp.full((_S, 64 - _A), _NEG, jnp.float32))
    o_ref[0] = jnp.concatenate(cols, axis=1)


def _vae_scores(slots, p):
    Bn = slots.shape[0]
    enc_w = p['vae_enc_w'].transpose(1, 0, 2).reshape(_D, _A * 32)
    enc_w = enc_w.astype(_BF)
    enc_b = p['vae_enc_b'].reshape(_A * 32)
    dec_w = p['vae_dec_w'].astype(_BF)
    full = lambda a: pl.BlockSpec(a.shape, lambda i: (0,) * a.ndim)
    return pl.pallas_call(
        _score_body,
        grid=(Bn,),
        in_specs=[pl.BlockSpec((1, _S, _D), lambda i: (i, 0, 0)),
                  full(enc_w), full(enc_b),
                  full(dec_w), full(p['vae_dec_b'])],
        out_specs=pl.BlockSpec((1, _S, 64), lambda i: (i, 0, 0)),
        out_shape=jax.ShapeDtypeStruct((Bn, _S, 64), jnp.float32),
    )(slots, enc_w, enc_b, dec_w, p['vae_dec_b'])


def _topk_sc_call(scores2d):
    R0 = scores2d.shape[0]
    NWROW = 16
    R = NWROW * 16
    sc = jnp.pad(scores2d, ((0, R - R0), (0, 0)))
    sc_t = sc.T.reshape(64, NWROW, 16).transpose(1, 0, 2)
    info = plsc.get_sparse_core_info()
    mesh = plsc.VectorSubcoreMesh(core_axis_name="c", subcore_axis_name="s")

    @functools.partial(
        pl.kernel, mesh=mesh,
        out_type=jax.ShapeDtypeStruct((NWROW, _K, 16), jnp.int32),
        scratch_types=[
            pltpu.VMEM((64, 16), jnp.float32),
            pltpu.VMEM((_K, 16), jnp.int32),
        ],
    )
    def _k(scores_hbm, out_hbm, sc_v, sel_v):
        wid = lax.axis_index("s") * info.num_cores + lax.axis_index("c")

        @pl.when(wid < NWROW)
        def _():
            pltpu.sync_copy(scores_hbm.at[wid], sc_v)
            b1 = jnp.full((16,), _NEG, jnp.float32)
            b2 = jnp.full((16,), _NEG, jnp.float32)
            b3 = jnp.full((16,), _NEG, jnp.float32)
            zero = jnp.zeros((16,), jnp.int32)
            i1, i2, i3 = zero, zero, zero
            for a in range(_A):
                v = sc_v[a, :]
                gt1 = v > b1
                gt2 = v > b2
                gt3 = v > b3
                av = jnp.full((16,), a, jnp.int32)
                b3 = jnp.where(gt2, b2, jnp.where(gt3, v, b3))
                i3 = jnp.where(gt2, i2, jnp.where(gt3, av, i3))
                b2 = jnp.where(gt1, b1, jnp.where(gt2, v, b2))
                i2 = jnp.where(gt1, i1, jnp.where(gt2, av, i2))
                b1 = jnp.where(gt1, v, b1)
                i1 = jnp.where(gt1, av, i1)
            sel_v[0, :] = i1
            sel_v[1, :] = i2
            sel_v[2, :] = i3
            pltpu.sync_copy(sel_v, out_hbm.at[wid])

    out = _k(sc_t)
    return out.transpose(0, 2, 1).reshape(R, _K)[:R0]


def _expert_dense_body(sel_ref, x_ref, in_w_ref, in_b_ref,
                       blk_w_ref, blk_b_ref, out_w_ref, out_b_ref, o_ref):
    a = pl.program_id(0)

    @pl.when(a == 0)
    def _():
        o_ref[...] = jnp.zeros_like(o_ref)

    x = x_ref[...]
    h = jnp.maximum(_dot(x, in_w_ref[0]) + in_b_ref[0], 0.0)
    for i in range(3):
        h = h + jnp.maximum(_dot(h, blk_w_ref[0, i]) + blk_b_ref[0, i], 0.0)
    out = _dot(h, out_w_ref[0]) + out_b_ref[0]
    mask = sel_ref[...] == a
    o_ref[...] += jnp.where(mask, out, 0.0)


def _expert_mlps(sel, slots, p):
    Bn = slots.shape[0]
    T = Bn * _S * _K
    x = jnp.broadcast_to(slots[:, :, None, :], (Bn, _S, _K, _D))
    x = x.reshape(T, _D)
    sel2 = sel.reshape(T, 1)
    in_w = p['ag_in_w'].astype(_BF)
    blk_w = p['ag_blk_w'].astype(_BF)
    out_w = p['ag_out_w'].astype(_BF)
    in_b = p['ag_in_b'].reshape(_A, 1, 256)
    out_b = p['ag_out_b'].reshape(_A, 1, _HD)
    return pl.pallas_call(
        _expert_dense_body,
        grid=(_A,),
        in_specs=[
            pl.BlockSpec((T, 1), lambda a: (0, 0)),
            pl.BlockSpec((T, _D), lambda a: (0, 0)),
            pl.BlockSpec((1, _D, 256), lambda a: (a, 0, 0)),
            pl.BlockSpec((1, 1, 256), lambda a: (a, 0, 0)),
            pl.BlockSpec((1, 3, 256, 256), lambda a: (a, 0, 0, 0)),
            pl.BlockSpec((1, 3, 256), lambda a: (a, 0, 0)),
            pl.BlockSpec((1, 256, _HD), lambda a: (a, 0, 0)),
            pl.BlockSpec((1, 1, _HD), lambda a: (a, 0, 0)),
        ],
        out_specs=pl.BlockSpec((T, _HD), lambda a: (0, 0)),
        out_shape=jax.ShapeDtypeStruct((T, _HD), jnp.float32),
    )(sel2, x, in_w, in_b, blk_w, p['ag_blk_b'],
      out_w, out_b)


def kernel(images, slot_noise, params):
    p = params
    Bn = images.shape[0]

    def _conv(x, w, b, stride):
        y = jax.lax.conv_general_dilated(
            x, w, (stride, stride), 'SAME',
            dimension_numbers=('NCHW', 'OIHW', 'NCHW'))
        return y + b[None, :, None, None]

    x = jax.nn.relu(_conv(images, p['conv1_w'], p['conv1_b'], 2))
    x = jax.nn.relu(_conv(x, p['conv2_w'], p['conv2_b'], 2))
    x = jax.nn.relu(_conv(x, p['conv3_w'], p['conv3_b'], 2))
    x = jax.nn.relu(_conv(x, p['conv4_w'], p['conv4_b'], 2))
    x = _conv(x, p['convo_w'], p['convo_b'], 1)
    feat = x.transpose(0, 2, 3, 1).reshape(Bn, -1, x.shape[1])
    inp = _ln(feat, p['ln_in_g'], p['ln_in_b'])
    kk = inp @ p['Wk']
    vv = inp @ p['Wv']

    slots = _attn_loop(kk, vv, slot_noise, p)
    scores = _vae_scores(slots, p)
    sel = _topk_sc_call(scores.reshape(Bn * _S, 64)).reshape(Bn, _S, _K)
    out = _expert_mlps(sel, slots, p)
    return out.reshape(Bn, _S * _K * _HD)
